# baseline (speedup 1.0000x reference)
import functools

import jax
import jax.numpy as jnp
from jax import lax
from jax.experimental import pallas as pl
from jax.experimental.pallas import tpu as pltpu
from jax.experimental.pallas import tpu_sc as plsc

N = 10000
E = 320000
D = 128

NC = 2
NS = 16
NW = NC * NS
NPAD = 10240
RPT = NPAD // NS
PADROW = 10016

C = 128
TPE = E // NW
NCHUNK = 80
TPE_P = NCHUNK * C

_mesh = plsc.VectorSubcoreMesh(core_axis_name="c", subcore_axis_name="s",
                               num_cores=NC, num_subcores=NS)


def _zero16():
    return jnp.zeros((16,), jnp.float32)


def _one16():
    return jnp.ones((16,), jnp.float32)


@functools.partial(
    pl.kernel,
    out_type=jax.ShapeDtypeStruct((NC, NPAD), jnp.float32),
    mesh=_mesh,
    scratch_types=[
        pltpu.VMEM_SHARED((NPAD,), jnp.float32),
        pltpu.VMEM((NCHUNK, C), jnp.int32),
        pltpu.VMEM((C,), jnp.float32),
        pltpu.VMEM((RPT,), jnp.float32),
    ],
)
def _sc_deg(dst3, deg_out, deg_sh, didx, ones_v, zv):
    c = lax.axis_index("c")
    s = lax.axis_index("s")
    wid = c * NS + s

    pltpu.sync_copy(dst3.at[wid], didx)

    def fill_z(i, _):
        zv[pl.ds(i * 16, 16)] = _zero16()
        return 0
    lax.fori_loop(0, RPT // 16, fill_z, 0)
    for i in range(C // 16):
        ones_v[pl.ds(i * 16, 16)] = _one16()

    pltpu.sync_copy(zv, deg_sh.at[pl.ds(s * RPT, RPT)])
    plsc.subcore_barrier()

    def step(j, _):
        pltpu.sync_copy(ones_v, deg_sh.at[didx.at[j]], add=True)
        return 0
    lax.fori_loop(0, NCHUNK, step, 0)
    plsc.subcore_barrier()

    pltpu.sync_copy(deg_sh.at[pl.ds(s * RPT, RPT)],
                    deg_out.at[c, pl.ds(s * RPT, RPT)])


@functools.partial(
    pl.kernel,
    out_type=jax.ShapeDtypeStruct((NC, NPAD, D), jnp.float32),
    mesh=_mesh,
    scratch_types=[
        pltpu.VMEM_SHARED((NPAD, D), jnp.float32),
        pltpu.VMEM((NCHUNK, C), jnp.int32),
        pltpu.VMEM((NCHUNK, C), jnp.int32),
        pltpu.VMEM((C, D), jnp.float32),
        pltpu.SemaphoreType.DMA,
    ],
)
def _sc_msg(hp_hbm, src3, dst3, acc_out, acc_sh, sidx, didx, rows, gsem):
    c = lax.axis_index("c")
    s = lax.axis_index("s")
    wid = c * NS + s

    pltpu.sync_copy(src3.at[wid], sidx)
    pltpu.sync_copy(dst3.at[wid], didx)

    def fill_z(i, _):
        for j in range(D // 16):
            rows[i, pl.ds(j * 16, 16)] = _zero16()
        return 0
    lax.fori_loop(0, C, fill_z, 0)

    def init(k, _):
        pltpu.sync_copy(rows, acc_sh.at[pl.ds(s * RPT + k * C, C)])
        return 0
    lax.fori_loop(0, RPT // C, init, 0)
    plsc.subcore_barrier()

    def step(j, _):
        pltpu.async_copy(hp_hbm.at[sidx.at[j]], rows, gsem).wait()
        pltpu.sync_copy(rows, acc_sh.at[didx.at[j]], add=True)
        return 0
    lax.fori_loop(0, NCHUNK, step, 0)
    plsc.subcore_barrier()

    pltpu.sync_copy(acc_sh.at[pl.ds(s * RPT, RPT)],
                    acc_out.at[c, pl.ds(s * RPT, RPT)])


_BLK = 1024
_GRID = (N + _BLK - 1) // _BLK


def _dis_col(deg_ref):
    return lax.rsqrt(deg_ref[0, :] + deg_ref[1, :] + 1.0)[:, None]


def _tc_pre_body(x_ref, w_ref, deg_ref, o_ref):
    h = jnp.dot(x_ref[:], w_ref[:], preferred_element_type=jnp.float32)
    o_ref[:] = h * _dis_col(deg_ref)


def _tc_mid_body(acc_ref, hp_ref, deg_ref, b_ref, w_ref, o_ref):
    dis = _dis_col(deg_ref)
    t = dis * (acc_ref[0] + acc_ref[1] + hp_ref[:]) + b_ref[:]
    t = jnp.maximum(t, 0.0)
    o_ref[:] = jnp.dot(t, w_ref[:], preferred_element_type=jnp.float32) * dis


def _tc_post_body(acc_ref, hp_ref, deg_ref, b_ref, o_ref):
    dis = _dis_col(deg_ref)
    o_ref[:] = jnp.maximum(dis * (acc_ref[0] + acc_ref[1] + hp_ref[:]) + b_ref[:],
                           0.0)


_x_spec = pl.BlockSpec((_BLK, D), lambda j: (j, 0))
_w_spec = pl.BlockSpec((D, D), lambda j: (0, 0))
_deg_spec = pl.BlockSpec((NC, _BLK), lambda j: (0, j))
_acc_spec = pl.BlockSpec((NC, _BLK, D), lambda j: (0, j, 0))
_b_spec = pl.BlockSpec((1, D), lambda j: (0, 0))
_out_sds = jax.ShapeDtypeStruct((N, D), jnp.float32)

_tc_pre = pl.pallas_call(
    _tc_pre_body, grid=(_GRID,),
    in_specs=[_x_spec, _w_spec, _deg_spec],
    out_specs=_x_spec, out_shape=_out_sds)

_tc_mid = pl.pallas_call(
    _tc_mid_body, grid=(_GRID,),
    in_specs=[_acc_spec, _x_spec, _deg_spec, _b_spec, _w_spec],
    out_specs=_x_spec, out_shape=_out_sds)

_tc_post = pl.pallas_call(
    _tc_post_body, grid=(_GRID,),
    in_specs=[_acc_spec, _x_spec, _deg_spec, _b_spec],
    out_specs=_x_spec, out_shape=_out_sds)


@jax.jit
def kernel(x, edge_index, W1, b1, W2, b2):
    src = edge_index[0].astype(jnp.int32).reshape(NW, TPE)
    dst = edge_index[1].astype(jnp.int32).reshape(NW, TPE)
    pad_n = TPE_P - TPE
    src3 = jnp.concatenate(
        [src, jnp.zeros((NW, pad_n), jnp.int32)], axis=1).reshape(NW, NCHUNK, C)
    dst3 = jnp.concatenate(
        [dst, jnp.full((NW, pad_n), PADROW, jnp.int32)],
        axis=1).reshape(NW, NCHUNK, C)
    b1r = b1.reshape(1, D)
    b2r = b2.reshape(1, D)

    deg2 = _sc_deg(dst3)
    hp1 = _tc_pre(x, W1, deg2)
    acc1 = _sc_msg(hp1, src3, dst3)
    hp2 = _tc_mid(acc1, hp1, deg2, b1r, W2)
    acc2 = _sc_msg(hp2, src3, dst3)
    return _tc_post(acc2, hp2, deg2, b2r)


# speedup vs baseline: 2.0371x; 2.0371x over previous
import functools

import jax
import jax.numpy as jnp
from jax import lax
from jax.experimental import pallas as pl
from jax.experimental.pallas import tpu as pltpu
from jax.experimental.pallas import tpu_sc as plsc

N = 10000
E = 320000
D = 128

NC = 2
NS = 16
NW = NC * NS
NPAD = 10240
RPT = NPAD // NS
PADROW = 10016

C = 128
TPE = E // NW
NCHUNK = 80
TPE_P = NCHUNK * C

_mesh = plsc.VectorSubcoreMesh(core_axis_name="c", subcore_axis_name="s",
                               num_cores=NC, num_subcores=NS)


def _zero16():
    return jnp.zeros((16,), jnp.float32)


def _one16():
    return jnp.ones((16,), jnp.float32)


@functools.partial(
    pl.kernel,
    out_type=jax.ShapeDtypeStruct((NC, NPAD), jnp.float32),
    mesh=_mesh,
    scratch_types=[
        pltpu.VMEM_SHARED((NPAD,), jnp.float32),
        pltpu.VMEM((NCHUNK, C), jnp.int32),
        pltpu.VMEM((C,), jnp.float32),
        pltpu.VMEM((RPT,), jnp.float32),
    ],
)
def _sc_deg(dst3, deg_out, deg_sh, didx, ones_v, zv):
    c = lax.axis_index("c")
    s = lax.axis_index("s")
    wid = c * NS + s

    pltpu.sync_copy(dst3.at[wid], didx)

    def fill_z(i, _):
        zv[pl.ds(i * 16, 16)] = _zero16()
        return 0
    lax.fori_loop(0, RPT // 16, fill_z, 0)
    for i in range(C // 16):
        ones_v[pl.ds(i * 16, 16)] = _one16()

    pltpu.sync_copy(zv, deg_sh.at[pl.ds(s * RPT, RPT)])
    plsc.subcore_barrier()

    def step(j, _):
        pltpu.sync_copy(ones_v, deg_sh.at[didx.at[j]], add=True)
        return 0
    lax.fori_loop(0, NCHUNK, step, 0)
    plsc.subcore_barrier()

    pltpu.sync_copy(deg_sh.at[pl.ds(s * RPT, RPT)],
                    deg_out.at[c, pl.ds(s * RPT, RPT)])


MC = 80
MCHUNK = TPE // MC

@functools.partial(
    pl.kernel,
    out_type=jax.ShapeDtypeStruct((NC, NPAD, D), jnp.float32),
    mesh=_mesh,
    scratch_types=[
        pltpu.VMEM_SHARED((NPAD, D), jnp.float32),
        [pltpu.VMEM((MC,), jnp.int32)] * 2,
        [pltpu.VMEM((MC,), jnp.int32)] * 2,
        pltpu.VMEM((MC, D), jnp.float32),
        [pltpu.SemaphoreType.DMA] * 2,
        [pltpu.SemaphoreType.DMA] * 2,
        pltpu.SemaphoreType.DMA,
    ],
)
def _sc_msg(hp_hbm, src_hbm, dst_hbm, acc_out, acc_sh, sidx, didx, rows,
            issems, idsems, gsem):
    c = lax.axis_index("c")
    s = lax.axis_index("s")
    wid = c * NS + s

    def fill_z(i, _):
        for j in range(D // 16):
            rows[i, pl.ds(j * 16, 16)] = _zero16()
        return 0
    lax.fori_loop(0, MC, fill_z, 0)

    def init(k, _):
        pltpu.sync_copy(rows, acc_sh.at[pl.ds(s * RPT + k * MC, MC)])
        return 0
    lax.fori_loop(0, RPT // MC, init, 0)
    plsc.subcore_barrier()

    def is_desc(j, b):
        base = wid * TPE + j * MC
        return pltpu.make_async_copy(src_hbm.at[pl.ds(base, MC)], sidx[b],
                                     issems[b])

    def id_desc(j, b):
        base = wid * TPE + j * MC
        return pltpu.make_async_copy(dst_hbm.at[pl.ds(base, MC)], didx[b],
                                     idsems[b])

    def half(j, b, prefetch):
        is_desc(j, b).wait()
        id_desc(j, b).wait()
        pltpu.async_copy(hp_hbm.at[sidx[b]], rows, gsem).wait()
        pltpu.sync_copy(rows, acc_sh.at[didx[b]], add=True)
        if prefetch:
            @pl.when(j + 2 < MCHUNK)
            def _():
                is_desc(j + 2, b).start()
                id_desc(j + 2, b).start()

    is_desc(0, 0).start()
    id_desc(0, 0).start()
    is_desc(1, 1).start()
    id_desc(1, 1).start()

    def step(g, _):
        j0 = g * 2
        half(j0, 0, True)
        half(j0 + 1, 1, True)
        return 0
    lax.fori_loop(0, MCHUNK // 2, step, 0)
    half(MCHUNK - 1, (MCHUNK - 1) % 2, False)
    plsc.subcore_barrier()

    pltpu.sync_copy(acc_sh.at[pl.ds(s * RPT, RPT)],
                    acc_out.at[c, pl.ds(s * RPT, RPT)])


_BLK = 1024
_GRID = (N + _BLK - 1) // _BLK


def _dis_col(deg_ref):
    return lax.rsqrt(deg_ref[0, :] + deg_ref[1, :] + 1.0)[:, None]


def _tc_pre_body(x_ref, w_ref, deg_ref, o_ref):
    h = jnp.dot(x_ref[:], w_ref[:], preferred_element_type=jnp.float32)
    o_ref[:] = h * _dis_col(deg_ref)


def _tc_mid_body(acc_ref, hp_ref, deg_ref, b_ref, w_ref, o_ref):
    dis = _dis_col(deg_ref)
    t = dis * (acc_ref[0] + acc_ref[1] + hp_ref[:]) + b_ref[:]
    t = jnp.maximum(t, 0.0)
    o_ref[:] = jnp.dot(t, w_ref[:], preferred_element_type=jnp.float32) * dis


def _tc_post_body(acc_ref, hp_ref, deg_ref, b_ref, o_ref):
    dis = _dis_col(deg_ref)
    o_ref[:] = jnp.maximum(dis * (acc_ref[0] + acc_ref[1] + hp_ref[:]) + b_ref[:],
                           0.0)


_x_spec = pl.BlockSpec((_BLK, D), lambda j: (j, 0))
_w_spec = pl.BlockSpec((D, D), lambda j: (0, 0))
_deg_spec = pl.BlockSpec((NC, _BLK), lambda j: (0, j))
_acc_spec = pl.BlockSpec((NC, _BLK, D), lambda j: (0, j, 0))
_b_spec = pl.BlockSpec((1, D), lambda j: (0, 0))
_out_sds = jax.ShapeDtypeStruct((N, D), jnp.float32)

_tc_pre = pl.pallas_call(
    _tc_pre_body, grid=(_GRID,),
    in_specs=[_x_spec, _w_spec, _deg_spec],
    out_specs=_x_spec, out_shape=_out_sds)

_tc_mid = pl.pallas_call(
    _tc_mid_body, grid=(_GRID,),
    in_specs=[_acc_spec, _x_spec, _deg_spec, _b_spec, _w_spec],
    out_specs=_x_spec, out_shape=_out_sds)

_tc_post = pl.pallas_call(
    _tc_post_body, grid=(_GRID,),
    in_specs=[_acc_spec, _x_spec, _deg_spec, _b_spec],
    out_specs=_x_spec, out_shape=_out_sds)


@jax.jit
def kernel(x, edge_index, W1, b1, W2, b2):
    src_f = edge_index[0].astype(jnp.int32)
    dst_f = edge_index[1].astype(jnp.int32)
    pad_n = TPE_P - TPE
    dst3 = jnp.concatenate(
        [dst_f.reshape(NW, TPE), jnp.full((NW, pad_n), PADROW, jnp.int32)],
        axis=1).reshape(NW, NCHUNK, C)
    b1r = b1.reshape(1, D)
    b2r = b2.reshape(1, D)

    deg2 = _sc_deg(dst3)
    hp1 = _tc_pre(x, W1, deg2)
    acc1 = _sc_msg(hp1, src_f, dst_f)
    hp2 = _tc_mid(acc1, hp1, deg2, b1r, W2)
    acc2 = _sc_msg(hp2, src_f, dst_f)
    return _tc_post(acc2, hp2, deg2, b2r)


# speedup vs baseline: 2.5785x; 1.2658x over previous
import functools

import jax
import jax.numpy as jnp
from jax import lax
from jax.experimental import pallas as pl
from jax.experimental.pallas import tpu as pltpu
from jax.experimental.pallas import tpu_sc as plsc

N = 10000
E = 320000
D = 128

NC = 2
NS = 16
NW = NC * NS
NPAD = 10240
RPT = NPAD // NS
PADROW = 10016

C = 128
TPE = E // NW
NCHUNK = 80
TPE_P = NCHUNK * C

_mesh = plsc.VectorSubcoreMesh(core_axis_name="c", subcore_axis_name="s",
                               num_cores=NC, num_subcores=NS)


def _zero16():
    return jnp.zeros((16,), jnp.float32)


def _one16():
    return jnp.ones((16,), jnp.float32)


@functools.partial(
    pl.kernel,
    out_type=jax.ShapeDtypeStruct((NC, NPAD), jnp.float32),
    mesh=_mesh,
    scratch_types=[
        pltpu.VMEM_SHARED((NPAD,), jnp.float32),
        pltpu.VMEM((NCHUNK, C), jnp.int32),
        pltpu.VMEM((C,), jnp.float32),
        pltpu.VMEM((RPT,), jnp.float32),
    ],
)
def _sc_deg(dst3, deg_out, deg_sh, didx, ones_v, zv):
    c = lax.axis_index("c")
    s = lax.axis_index("s")
    wid = c * NS + s

    pltpu.sync_copy(dst3.at[wid], didx)

    def fill_z(i, _):
        zv[pl.ds(i * 16, 16)] = _zero16()
        return 0
    lax.fori_loop(0, RPT // 16, fill_z, 0)
    for i in range(C // 16):
        ones_v[pl.ds(i * 16, 16)] = _one16()

    pltpu.sync_copy(zv, deg_sh.at[pl.ds(s * RPT, RPT)])
    plsc.subcore_barrier()

    def step(j, _):
        pltpu.sync_copy(ones_v, deg_sh.at[didx.at[j]], add=True)
        return 0
    lax.fori_loop(0, NCHUNK, step, 0)
    plsc.subcore_barrier()

    pltpu.sync_copy(deg_sh.at[pl.ds(s * RPT, RPT)],
                    deg_out.at[c, pl.ds(s * RPT, RPT)])


MC = 80
MCHUNK = TPE // MC

@functools.partial(
    pl.kernel,
    out_type=jax.ShapeDtypeStruct((NC, NPAD, D), jnp.float32),
    mesh=_mesh,
    scratch_types=[
        pltpu.VMEM_SHARED((NPAD, D), jnp.float32),
        [pltpu.VMEM((MC,), jnp.int32)] * 2,
        [pltpu.VMEM((MC,), jnp.int32)] * 2,
        [pltpu.VMEM((MC, D), jnp.float32)] * 2,
        [pltpu.SemaphoreType.DMA] * 2,
        [pltpu.SemaphoreType.DMA] * 2,
        [pltpu.SemaphoreType.DMA] * 2,
    ],
)
def _sc_msg(hp_hbm, src_hbm, dst_hbm, acc_out, acc_sh, sidx, didx, rows,
            issems, idsems, gsems):
    c = lax.axis_index("c")
    s = lax.axis_index("s")
    wid = c * NS + s

    def fill_z(i, _):
        for j in range(D // 16):
            rows[0][i, pl.ds(j * 16, 16)] = _zero16()
        return 0
    lax.fori_loop(0, MC, fill_z, 0)

    def init(k, _):
        pltpu.sync_copy(rows[0], acc_sh.at[pl.ds(s * RPT + k * MC, MC)])
        return 0
    lax.fori_loop(0, RPT // MC, init, 0)
    plsc.subcore_barrier()

    def is_desc(j, b):
        base = wid * TPE + j * MC
        return pltpu.make_async_copy(src_hbm.at[pl.ds(base, MC)], sidx[b],
                                     issems[b])

    def id_desc(j, b):
        base = wid * TPE + j * MC
        return pltpu.make_async_copy(dst_hbm.at[pl.ds(base, MC)], didx[b],
                                     idsems[b])

    def g_desc(j, b):
        return pltpu.make_async_copy(hp_hbm.at[sidx[b]], rows[b], gsems[b])

    def half(j, b):
        g_desc(j, b).wait()

        @pl.when(j + 1 < MCHUNK)
        def _():
            is_desc(j + 1, 1 - b).wait()
            g_desc(j + 1, 1 - b).start()
        id_desc(j, b).wait()
        pltpu.sync_copy(rows[b], acc_sh.at[didx[b]], add=True)

        @pl.when(j + 2 < MCHUNK)
        def _():
            is_desc(j + 2, b).start()
            id_desc(j + 2, b).start()

    is_desc(0, 0).start()
    id_desc(0, 0).start()
    is_desc(1, 1).start()
    id_desc(1, 1).start()
    is_desc(0, 0).wait()
    g_desc(0, 0).start()

    def step(g, _):
        j0 = g * 2
        half(j0, 0)
        half(j0 + 1, 1)
        return 0
    lax.fori_loop(0, MCHUNK // 2, step, 0)
    half(MCHUNK - 1, (MCHUNK - 1) % 2)
    plsc.subcore_barrier()

    pltpu.sync_copy(acc_sh.at[pl.ds(s * RPT, RPT)],
                    acc_out.at[c, pl.ds(s * RPT, RPT)])


_BLK = 1024
_GRID = (N + _BLK - 1) // _BLK


def _dis_col(deg_ref):
    return lax.rsqrt(deg_ref[0, :] + deg_ref[1, :] + 1.0)[:, None]


def _tc_pre_body(x_ref, w_ref, deg_ref, o_ref):
    h = jnp.dot(x_ref[:], w_ref[:], preferred_element_type=jnp.float32)
    o_ref[:] = h * _dis_col(deg_ref)


def _tc_mid_body(acc_ref, hp_ref, deg_ref, b_ref, w_ref, o_ref):
    dis = _dis_col(deg_ref)
    t = dis * (acc_ref[0] + acc_ref[1] + hp_ref[:]) + b_ref[:]
    t = jnp.maximum(t, 0.0)
    o_ref[:] = jnp.dot(t, w_ref[:], preferred_element_type=jnp.float32) * dis


def _tc_post_body(acc_ref, hp_ref, deg_ref, b_ref, o_ref):
    dis = _dis_col(deg_ref)
    o_ref[:] = jnp.maximum(dis * (acc_ref[0] + acc_ref[1] + hp_ref[:]) + b_ref[:],
                           0.0)


_x_spec = pl.BlockSpec((_BLK, D), lambda j: (j, 0))
_w_spec = pl.BlockSpec((D, D), lambda j: (0, 0))
_deg_spec = pl.BlockSpec((NC, _BLK), lambda j: (0, j))
_acc_spec = pl.BlockSpec((NC, _BLK, D), lambda j: (0, j, 0))
_b_spec = pl.BlockSpec((1, D), lambda j: (0, 0))
_out_sds = jax.ShapeDtypeStruct((N, D), jnp.float32)

_tc_pre = pl.pallas_call(
    _tc_pre_body, grid=(_GRID,),
    in_specs=[_x_spec, _w_spec, _deg_spec],
    out_specs=_x_spec, out_shape=_out_sds)

_tc_mid = pl.pallas_call(
    _tc_mid_body, grid=(_GRID,),
    in_specs=[_acc_spec, _x_spec, _deg_spec, _b_spec, _w_spec],
    out_specs=_x_spec, out_shape=_out_sds)

_tc_post = pl.pallas_call(
    _tc_post_body, grid=(_GRID,),
    in_specs=[_acc_spec, _x_spec, _deg_spec, _b_spec],
    out_specs=_x_spec, out_shape=_out_sds)


@jax.jit
def kernel(x, edge_index, W1, b1, W2, b2):
    src_f = edge_index[0].astype(jnp.int32)
    dst_f = edge_index[1].astype(jnp.int32)
    pad_n = TPE_P - TPE
    dst3 = jnp.concatenate(
        [dst_f.reshape(NW, TPE), jnp.full((NW, pad_n), PADROW, jnp.int32)],
        axis=1).reshape(NW, NCHUNK, C)
    b1r = b1.reshape(1, D)
    b2r = b2.reshape(1, D)

    deg2 = _sc_deg(dst3)
    hp1 = _tc_pre(x, W1, deg2)
    acc1 = _sc_msg(hp1, src_f, dst_f)
    hp2 = _tc_mid(acc1, hp1, deg2, b1r, W2)
    acc2 = _sc_msg(hp2, src_f, dst_f)
    return _tc_post(acc2, hp2, deg2, b2r)


# speedup vs baseline: 2.9511x; 1.1445x over previous
import functools

import jax
import jax.numpy as jnp
from jax import lax
from jax.experimental import pallas as pl
from jax.experimental.pallas import tpu as pltpu
from jax.experimental.pallas import tpu_sc as plsc

N = 10000
E = 320000
D = 128

NC = 2
NS = 16
NW = NC * NS
NPAD = 10240
RPT = NPAD // NS
PADROW = 10016

C = 128
TPE = E // NW
NCHUNK = 80
TPE_P = NCHUNK * C

_mesh = plsc.VectorSubcoreMesh(core_axis_name="c", subcore_axis_name="s",
                               num_cores=NC, num_subcores=NS)


def _zero16():
    return jnp.zeros((16,), jnp.float32)


def _one16():
    return jnp.ones((16,), jnp.float32)


@functools.partial(
    pl.kernel,
    out_type=jax.ShapeDtypeStruct((NC, NPAD), jnp.float32),
    mesh=_mesh,
    scratch_types=[
        pltpu.VMEM_SHARED((NPAD,), jnp.float32),
        pltpu.VMEM((NCHUNK, C), jnp.int32),
        pltpu.VMEM((C,), jnp.float32),
        pltpu.VMEM((RPT,), jnp.float32),
    ],
)
def _sc_deg(dst3, deg_out, deg_sh, didx, ones_v, zv):
    c = lax.axis_index("c")
    s = lax.axis_index("s")
    wid = c * NS + s

    pltpu.sync_copy(dst3.at[wid], didx)

    def fill_z(i, _):
        zv[pl.ds(i * 16, 16)] = _zero16()
        return 0
    lax.fori_loop(0, RPT // 16, fill_z, 0)
    for i in range(C // 16):
        ones_v[pl.ds(i * 16, 16)] = _one16()

    pltpu.sync_copy(zv, deg_sh.at[pl.ds(s * RPT, RPT)])
    plsc.subcore_barrier()

    def step(j, _):
        pltpu.sync_copy(ones_v, deg_sh.at[didx.at[j]], add=True)
        return 0
    lax.fori_loop(0, NCHUNK, step, 0)
    plsc.subcore_barrier()

    pltpu.sync_copy(deg_sh.at[pl.ds(s * RPT, RPT)],
                    deg_out.at[c, pl.ds(s * RPT, RPT)])


MC = 80
MCHUNK = TPE // MC

@functools.partial(
    pl.kernel,
    out_type=jax.ShapeDtypeStruct((NC, NPAD, D), jnp.float32),
    mesh=_mesh,
    scratch_types=[
        pltpu.VMEM_SHARED((NPAD, D), jnp.float32),
        [pltpu.VMEM((MC,), jnp.int32)] * 3,
        [pltpu.VMEM((MC,), jnp.int32)] * 3,
        [pltpu.VMEM((MC, D), jnp.float32)] * 3,
        [pltpu.SemaphoreType.DMA] * 3,
        [pltpu.SemaphoreType.DMA] * 3,
        [pltpu.SemaphoreType.DMA] * 3,
    ],
)
def _sc_msg(hp_hbm, src_hbm, dst_hbm, acc_out, acc_sh, sidx, didx, rows,
            issems, idsems, gsems):
    c = lax.axis_index("c")
    s = lax.axis_index("s")
    wid = c * NS + s

    def fill_z(i, _):
        for j in range(D // 16):
            rows[0][i, pl.ds(j * 16, 16)] = _zero16()
        return 0
    lax.fori_loop(0, MC, fill_z, 0)

    def init(k, _):
        pltpu.sync_copy(rows[0], acc_sh.at[pl.ds(s * RPT + k * MC, MC)])
        return 0
    lax.fori_loop(0, RPT // MC, init, 0)
    plsc.subcore_barrier()

    def is_desc(j, b):
        base = wid * TPE + j * MC
        return pltpu.make_async_copy(src_hbm.at[pl.ds(base, MC)], sidx[b],
                                     issems[b])

    def id_desc(j, b):
        base = wid * TPE + j * MC
        return pltpu.make_async_copy(dst_hbm.at[pl.ds(base, MC)], didx[b],
                                     idsems[b])

    def g_desc(j, b):
        return pltpu.make_async_copy(hp_hbm.at[sidx[b]], rows[b], gsems[b])

    def half(j, b):
        g_desc(j, b).wait()

        @pl.when(j + 2 < MCHUNK)
        def _():
            is_desc(j + 2, (b + 2) % 3).wait()
            g_desc(j + 2, (b + 2) % 3).start()
        id_desc(j, b).wait()
        pltpu.sync_copy(rows[b], acc_sh.at[didx[b]], add=True)

        @pl.when(j + 3 < MCHUNK)
        def _():
            is_desc(j + 3, b).start()
            id_desc(j + 3, b).start()

    for b0 in range(3):
        is_desc(b0, b0).start()
        id_desc(b0, b0).start()
    is_desc(0, 0).wait()
    g_desc(0, 0).start()
    is_desc(1, 1).wait()
    g_desc(1, 1).start()

    def step(g, _):
        j0 = g * 3
        half(j0, 0)
        half(j0 + 1, 1)
        half(j0 + 2, 2)
        return 0
    lax.fori_loop(0, MCHUNK // 3, step, 0)
    half(MCHUNK - 2, (MCHUNK - 2) % 3)
    half(MCHUNK - 1, (MCHUNK - 1) % 3)
    plsc.subcore_barrier()

    pltpu.sync_copy(acc_sh.at[pl.ds(s * RPT, RPT)],
                    acc_out.at[c, pl.ds(s * RPT, RPT)])


_BLK = 1024
_GRID = (N + _BLK - 1) // _BLK


def _dis_col(deg_ref):
    return lax.rsqrt(deg_ref[0, :] + deg_ref[1, :] + 1.0)[:, None]


def _tc_pre_body(x_ref, w_ref, deg_ref, o_ref):
    h = jnp.dot(x_ref[:], w_ref[:], preferred_element_type=jnp.float32)
    o_ref[:] = h * _dis_col(deg_ref)


def _tc_mid_body(acc_ref, hp_ref, deg_ref, b_ref, w_ref, o_ref):
    dis = _dis_col(deg_ref)
    t = dis * (acc_ref[0] + acc_ref[1] + hp_ref[:]) + b_ref[:]
    t = jnp.maximum(t, 0.0)
    o_ref[:] = jnp.dot(t, w_ref[:], preferred_element_type=jnp.float32) * dis


def _tc_post_body(acc_ref, hp_ref, deg_ref, b_ref, o_ref):
    dis = _dis_col(deg_ref)
    o_ref[:] = jnp.maximum(dis * (acc_ref[0] + acc_ref[1] + hp_ref[:]) + b_ref[:],
                           0.0)


_x_spec = pl.BlockSpec((_BLK, D), lambda j: (j, 0))
_w_spec = pl.BlockSpec((D, D), lambda j: (0, 0))
_deg_spec = pl.BlockSpec((NC, _BLK), lambda j: (0, j))
_acc_spec = pl.BlockSpec((NC, _BLK, D), lambda j: (0, j, 0))
_b_spec = pl.BlockSpec((1, D), lambda j: (0, 0))
_out_sds = jax.ShapeDtypeStruct((N, D), jnp.float32)

_tc_pre = pl.pallas_call(
    _tc_pre_body, grid=(_GRID,),
    in_specs=[_x_spec, _w_spec, _deg_spec],
    out_specs=_x_spec, out_shape=_out_sds)

_tc_mid = pl.pallas_call(
    _tc_mid_body, grid=(_GRID,),
    in_specs=[_acc_spec, _x_spec, _deg_spec, _b_spec, _w_spec],
    out_specs=_x_spec, out_shape=_out_sds)

_tc_post = pl.pallas_call(
    _tc_post_body, grid=(_GRID,),
    in_specs=[_acc_spec, _x_spec, _deg_spec, _b_spec],
    out_specs=_x_spec, out_shape=_out_sds)


@jax.jit
def kernel(x, edge_index, W1, b1, W2, b2):
    src_f = edge_index[0].astype(jnp.int32)
    dst_f = edge_index[1].astype(jnp.int32)
    pad_n = TPE_P - TPE
    dst3 = jnp.concatenate(
        [dst_f.reshape(NW, TPE), jnp.full((NW, pad_n), PADROW, jnp.int32)],
        axis=1).reshape(NW, NCHUNK, C)
    b1r = b1.reshape(1, D)
    b2r = b2.reshape(1, D)

    deg2 = _sc_deg(dst3)
    hp1 = _tc_pre(x, W1, deg2)
    acc1 = _sc_msg(hp1, src_f, dst_f)
    hp2 = _tc_mid(acc1, hp1, deg2, b1r, W2)
    acc2 = _sc_msg(hp2, src_f, dst_f)
    return _tc_post(acc2, hp2, deg2, b2r)
